# baseline (speedup 1.0000x reference)
import jax
import jax.numpy as jnp
from jax import lax
from jax.experimental import pallas as pl
from jax.experimental.pallas import tpu as pltpu

EPS = 1e-06
TOP_M = 64
KPAD = 128
N_STEPS = 16
LG_STEPS = 8
SEL_START = 9
SEL_SCHED = (10, 10, 10, 10, 10, 10, 4)


def _fused(f1_ref, f2_ref, lg_ref, y_ref, ylg_ref, out_ref,
           cr_ref, cc_ref, s1_ref, q1_ref, s2_ref, q2_ref, p_ref,
           amat_ref, seli_ref, selj_ref, avec_ref):
    step = pl.program_id(0)
    bB = f1_ref.shape[0]
    bL = lg_ref.shape[0]
    K = lg_ref.shape[1]
    dn = (((0,), (0,)), ((), ()))

    @pl.when(step == 0)
    def _init():
        cr_ref[...] = jnp.zeros_like(cr_ref)
        cc_ref[...] = jnp.zeros_like(cc_ref)
        s1_ref[...] = jnp.zeros_like(s1_ref)
        q1_ref[...] = jnp.zeros_like(q1_ref)
        s2_ref[...] = jnp.zeros_like(s2_ref)
        q2_ref[...] = jnp.zeros_like(q2_ref)
        p_ref[...] = jnp.zeros_like(p_ref)

    z1 = f1_ref[...]
    z2 = f2_ref[...]
    yb = y_ref[...]
    ks = jax.lax.broadcasted_iota(jnp.int32, (bB, KPAD), 1)
    oh = (yb == ks).astype(jnp.float32)
    s1_ref[...] += jax.lax.dot_general(oh, z1, dn, preferred_element_type=jnp.float32)
    q1_ref[...] += jax.lax.dot_general(oh, z1 * z1, dn, preferred_element_type=jnp.float32)
    s2_ref[...] += jax.lax.dot_general(oh, z2, dn, preferred_element_type=jnp.float32)
    q2_ref[...] += jax.lax.dot_general(oh, z2 * z2, dn, preferred_element_type=jnp.float32)

    @pl.when(step < LG_STEPS)
    def _logits():
        y4 = ylg_ref[...].reshape(bL, 1)
        ks4 = jax.lax.broadcasted_iota(jnp.int32, (bL, KPAD), 1)
        oh4 = (y4 == ks4).astype(jnp.float32)
        cr_ref[...] += jnp.sum(oh4, axis=0, keepdims=True)
        cc_ref[...] += jax.lax.dot_general(
            oh4, jnp.ones((bL, 1), jnp.float32), dn,
            preferred_element_type=jnp.float32)
        lg = lg_ref[...]
        m = jnp.max(lg, axis=1, keepdims=True)
        e = jnp.exp(lg - m)
        p = e / jnp.sum(e, axis=1, keepdims=True)
        p_ref[...] += jax.lax.dot_general(oh4, p, dn,
                                          preferred_element_type=jnp.float32)

    @pl.when(step == LG_STEPS)
    def _build_amat():
        c_row = cr_ref[...]
        c_col = cc_ref[...]
        rinv = 1.0 / jnp.maximum(c_col, 1.0)
        valid_row = (c_row >= 2.0)
        valid_col = (c_col >= 2.0)
        mean_p = p_ref[...] * rinv
        ei = jax.lax.broadcasted_iota(jnp.int32, (K, KPAD), 0)
        ej = jax.lax.broadcasted_iota(jnp.int32, (K, KPAD), 1)
        pad_eye = (ei == ej).astype(jnp.float32)
        mp = jax.lax.dot_general(mean_p, pad_eye, (((1,), (0,)), ((), ())),
                                 preferred_element_type=jnp.float32)
        alpha = 0.5 * (mp + mp.T)
        ri = jax.lax.broadcasted_iota(jnp.int32, (KPAD, KPAD), 0)
        cj = jax.lax.broadcasted_iota(jnp.int32, (KPAD, KPAD), 1)
        keep = jnp.logical_and(cj > ri, jnp.logical_and(valid_col, valid_row))
        amat_ref[...] = jnp.where(keep, alpha, 0.0)

    base = 0
    for si, n_it in enumerate(SEL_SCHED):
        s = SEL_START + si

        @pl.when(step == s)
        def _select(base=base, n_it=n_it):
            ri = jax.lax.broadcasted_iota(jnp.int32, (KPAD, KPAD), 0)
            cj = jax.lax.broadcasted_iota(jnp.int32, (KPAD, KPAD), 1)
            fidx = ri * KPAD + cj
            fidxT = cj * KPAD + ri
            amat = amat_ref[...]
            for it in range(n_it):
                g = base + it
                a = jnp.max(amat, axis=(0, 1), keepdims=True)
                idx = jnp.min(jnp.where(amat == a, fidx, jnp.int32(2 ** 30)),
                              axis=(0, 1), keepdims=True)
                mask = fidx == idx
                jind = jnp.sum(mask.astype(jnp.float32), axis=0, keepdims=True)
                iind = jnp.sum((fidxT == idx).astype(jnp.float32), axis=0,
                               keepdims=True)
                seli_ref[pl.ds(g, 1), :] = iind
                selj_ref[pl.ds(g, 1), :] = jind
                avec_ref[pl.ds(g, 1), :] = a
                amat = jnp.where(mask, -1.0, amat)
            amat_ref[...] = amat

        base += n_it

    @pl.when(step == N_STEPS - 1)
    def _finale():
        c_row = cr_ref[...]
        c_col = cc_ref[...]
        rinv = 1.0 / jnp.maximum(c_col, 1.0)
        valid_row = (c_row >= 2.0)
        num_valid = jnp.sum(valid_row.astype(jnp.float32))
        kept = jnp.sum(jnp.where(valid_row, c_row, 0.0))
        layer_valid = jnp.logical_and(num_valid >= 2.0, kept >= 4.0)

        sel_i = seli_ref[...]
        sel_j = selj_ref[...]
        sel_d = sel_i - sel_j
        avec = avec_ref[...]
        dnm = (((1,), (0,)), ((), ()))

        def layer(s_ref, q_ref):
            mu = s_ref[...] * rinv
            var = jnp.maximum(q_ref[...] * rinv - mu * mu, EPS)
            d = jax.lax.dot_general(sel_d, mu, dnm, preferred_element_type=jnp.float32)
            vi = jax.lax.dot_general(sel_i, var, dnm, preferred_element_type=jnp.float32)
            vj = jax.lax.dot_general(sel_j, var, dnm, preferred_element_type=jnp.float32)
            va = 0.5 * (vi + vj) + EPS
            t1 = 0.125 * jnp.sum(d * d / va, axis=1, keepdims=True)
            t2 = 0.25 * jnp.sum(
                jnp.log(va * va / ((vi + EPS) * (vj + EPS))), axis=1, keepdims=True)
            dm = jnp.maximum(t1 + t2, 0.0)
            rho = jnp.exp(-dm)
            return jnp.sum(avec * rho)

        num1 = layer(s1_ref, q1_ref)
        num2 = layer(s2_ref, q2_ref)
        den = jnp.maximum(jnp.sum(avec), EPS)
        total = (num1 + num2) / den
        out_ref[...] = jnp.full((1, 1), jnp.where(layer_valid, total * 0.5, 0.0),
                                jnp.float32)


def kernel(feat_layer1, feat_layer2, logits, y):
    B, D = feat_layer1.shape
    K = logits.shape[1]
    bB = B // N_STEPS
    bL = B // LG_STEPS

    yi = y.astype(jnp.int32)
    y2 = yi.reshape(B, 1)
    y3 = yi.reshape(LG_STEPS, bL, 1)

    out = pl.pallas_call(
        _fused,
        grid=(N_STEPS,),
        in_specs=[
            pl.BlockSpec((bB, D), lambda i: (i, 0)),
            pl.BlockSpec((bB, D), lambda i: (i, 0)),
            pl.BlockSpec((bL, K), lambda i: (jnp.minimum(i, LG_STEPS - 1), 0)),
            pl.BlockSpec((bB, 1), lambda i: (i, 0)),
            pl.BlockSpec((1, bL, 1), lambda i: (jnp.minimum(i, LG_STEPS - 1), 0, 0)),
        ],
        out_specs=pl.BlockSpec((1, 1), lambda i: (0, 0)),
        out_shape=jax.ShapeDtypeStruct((1, 1), jnp.float32),
        scratch_shapes=[
            pltpu.VMEM((1, KPAD), jnp.float32),
            pltpu.VMEM((KPAD, 1), jnp.float32),
            pltpu.VMEM((KPAD, D), jnp.float32),
            pltpu.VMEM((KPAD, D), jnp.float32),
            pltpu.VMEM((KPAD, D), jnp.float32),
            pltpu.VMEM((KPAD, D), jnp.float32),
            pltpu.VMEM((KPAD, K), jnp.float32),
            pltpu.VMEM((KPAD, KPAD), jnp.float32),
            pltpu.VMEM((TOP_M, KPAD), jnp.float32),
            pltpu.VMEM((TOP_M, KPAD), jnp.float32),
            pltpu.VMEM((TOP_M, 1), jnp.float32),
        ],
    )(feat_layer1.astype(jnp.float32), feat_layer2.astype(jnp.float32),
      logits.astype(jnp.float32), y2, y3)
    return out.reshape(())


# speedup vs baseline: 1.1619x; 1.1619x over previous
import jax
import jax.numpy as jnp
from jax import lax
from jax.experimental import pallas as pl
from jax.experimental.pallas import tpu as pltpu

EPS = 1e-06
TOP_M = 64
KPAD = 128


def _fused(f1_ref, f2_ref, lg_ref, y_ref, out_ref,
           cr_ref, cc_ref, s1_ref, q1_ref, s2_ref, q2_ref, p_ref,
           seli_ref, selj_ref, avec_ref):
    step = pl.program_id(0)
    nsteps = pl.num_programs(0)

    @pl.when(step == 0)
    def _init():
        cr_ref[...] = jnp.zeros_like(cr_ref)
        cc_ref[...] = jnp.zeros_like(cc_ref)
        s1_ref[...] = jnp.zeros_like(s1_ref)
        q1_ref[...] = jnp.zeros_like(q1_ref)
        s2_ref[...] = jnp.zeros_like(s2_ref)
        q2_ref[...] = jnp.zeros_like(q2_ref)
        p_ref[...] = jnp.zeros_like(p_ref)

    z1 = f1_ref[...]
    z2 = f2_ref[...]
    lg = lg_ref[...]
    yb = y_ref[...]

    bB = z1.shape[0]
    ks = jax.lax.broadcasted_iota(jnp.int32, (bB, KPAD), 1)
    oh = (yb == ks).astype(jnp.float32)

    dn = (((0,), (0,)), ((), ()))
    cr_ref[...] += jnp.sum(oh, axis=0, keepdims=True)
    cc_ref[...] += jax.lax.dot_general(
        oh, jnp.ones((bB, 1), jnp.float32), dn, preferred_element_type=jnp.float32)
    s1_ref[...] += jax.lax.dot_general(oh, z1, dn, preferred_element_type=jnp.float32)
    q1_ref[...] += jax.lax.dot_general(oh, z1 * z1, dn, preferred_element_type=jnp.float32)
    s2_ref[...] += jax.lax.dot_general(oh, z2, dn, preferred_element_type=jnp.float32)
    q2_ref[...] += jax.lax.dot_general(oh, z2 * z2, dn, preferred_element_type=jnp.float32)

    m = jnp.max(lg, axis=1, keepdims=True)
    e = jnp.exp(lg - m)
    p = e / jnp.sum(e, axis=1, keepdims=True)
    p_ref[...] += jax.lax.dot_general(oh, p, dn, preferred_element_type=jnp.float32)

    @pl.when(step == nsteps - 1)
    def _finale():
        K = p_ref.shape[1]
        c_row = cr_ref[...]
        c_col = cc_ref[...]
        rinv = 1.0 / jnp.maximum(c_col, 1.0)

        valid_row = (c_row >= 2.0)
        valid_col = (c_col >= 2.0)
        num_valid = jnp.sum(valid_row.astype(jnp.float32))
        kept = jnp.sum(jnp.where(valid_row, c_row, 0.0))
        layer_valid = jnp.logical_and(num_valid >= 2.0, kept >= 4.0)

        mu1 = s1_ref[...] * rinv
        var1 = jnp.maximum(q1_ref[...] * rinv - mu1 * mu1, EPS)
        mu2 = s2_ref[...] * rinv
        var2 = jnp.maximum(q2_ref[...] * rinv - mu2 * mu2, EPS)

        mean_p = p_ref[...] * rinv
        ei = jax.lax.broadcasted_iota(jnp.int32, (K, KPAD), 0)
        ej = jax.lax.broadcasted_iota(jnp.int32, (K, KPAD), 1)
        pad_eye = (ei == ej).astype(jnp.float32)
        mp = jax.lax.dot_general(mean_p, pad_eye, (((1,), (0,)), ((), ())),
                                 preferred_element_type=jnp.float32)
        alpha = 0.5 * (mp + mp.T)

        ri = jax.lax.broadcasted_iota(jnp.int32, (KPAD, KPAD), 0)
        cj = jax.lax.broadcasted_iota(jnp.int32, (KPAD, KPAD), 1)
        keep = jnp.logical_and(cj > ri, jnp.logical_and(valid_col, valid_row))
        amat = jnp.where(keep, alpha, 0.0)
        fidx = ri * KPAD + cj
        fidxT = cj * KPAD + ri

        for p_ in range(TOP_M):
            a = jnp.max(amat, axis=(0, 1), keepdims=True)
            idx = jnp.min(jnp.where(amat == a, fidx, jnp.int32(2 ** 30)),
                          axis=(0, 1), keepdims=True)
            mask = fidx == idx
            jind = jnp.sum(mask.astype(jnp.float32), axis=0, keepdims=True)
            iind = jnp.sum((fidxT == idx).astype(jnp.float32), axis=0,
                           keepdims=True)
            seli_ref[pl.ds(p_, 1), :] = iind
            selj_ref[pl.ds(p_, 1), :] = jind
            avec_ref[pl.ds(p_, 1), :] = a
            amat = jnp.where(mask, -1.0, amat)

        sel_i = seli_ref[...]
        sel_j = selj_ref[...]
        sel_d = sel_i - sel_j
        avec = avec_ref[...]
        dnm = (((1,), (0,)), ((), ()))

        def layer(mu, var):
            d = jax.lax.dot_general(sel_d, mu, dnm, preferred_element_type=jnp.float32)
            vi = jax.lax.dot_general(sel_i, var, dnm, preferred_element_type=jnp.float32)
            vj = jax.lax.dot_general(sel_j, var, dnm, preferred_element_type=jnp.float32)
            va = 0.5 * (vi + vj) + EPS
            t1 = 0.125 * jnp.sum(d * d / va, axis=1, keepdims=True)
            t2 = 0.25 * jnp.sum(
                jnp.log(va * va / ((vi + EPS) * (vj + EPS))), axis=1, keepdims=True)
            dm = jnp.maximum(t1 + t2, 0.0)
            rho = jnp.exp(-dm)
            return jnp.sum(avec * rho)

        num1 = layer(mu1, var1)
        num2 = layer(mu2, var2)
        den = jnp.maximum(jnp.sum(avec), EPS)
        total = (num1 + num2) / den
        out_ref[...] = jnp.full((1, 1), jnp.where(layer_valid, total * 0.5, 0.0),
                                jnp.float32)


def kernel(feat_layer1, feat_layer2, logits, y):
    B, D = feat_layer1.shape
    K = logits.shape[1]
    bB = 1024
    grid = B // bB

    y2 = y.astype(jnp.int32).reshape(B, 1)

    out = pl.pallas_call(
        _fused,
        grid=(grid,),
        in_specs=[
            pl.BlockSpec((bB, D), lambda i: (i, 0)),
            pl.BlockSpec((bB, D), lambda i: (i, 0)),
            pl.BlockSpec((bB, K), lambda i: (i, 0)),
            pl.BlockSpec((bB, 1), lambda i: (i, 0)),
        ],
        out_specs=pl.BlockSpec((1, 1), lambda i: (0, 0)),
        out_shape=jax.ShapeDtypeStruct((1, 1), jnp.float32),
        scratch_shapes=[
            pltpu.VMEM((1, KPAD), jnp.float32),
            pltpu.VMEM((KPAD, 1), jnp.float32),
            pltpu.VMEM((KPAD, D), jnp.float32),
            pltpu.VMEM((KPAD, D), jnp.float32),
            pltpu.VMEM((KPAD, D), jnp.float32),
            pltpu.VMEM((KPAD, D), jnp.float32),
            pltpu.VMEM((KPAD, K), jnp.float32),
            pltpu.VMEM((TOP_M, KPAD), jnp.float32),
            pltpu.VMEM((TOP_M, KPAD), jnp.float32),
            pltpu.VMEM((TOP_M, 1), jnp.float32),
        ],
    )(feat_layer1.astype(jnp.float32), feat_layer2.astype(jnp.float32),
      logits.astype(jnp.float32), y2)
    return out.reshape(())


# speedup vs baseline: 1.2350x; 1.0629x over previous
import jax
import jax.numpy as jnp
from jax import lax
from jax.experimental import pallas as pl
from jax.experimental.pallas import tpu as pltpu

EPS = 1e-06
TOP_M = 64
KPAD = 128


def _fused(f1_ref, f2_ref, lg_ref, y_ref, out_ref,
           cr_ref, cc_ref, s1_ref, q1_ref, s2_ref, q2_ref, p_ref,
           seli_ref, selj_ref, avec_ref):
    step = pl.program_id(0)
    nsteps = pl.num_programs(0)

    @pl.when(step == 0)
    def _init():
        cr_ref[...] = jnp.zeros_like(cr_ref)
        cc_ref[...] = jnp.zeros_like(cc_ref)
        s1_ref[...] = jnp.zeros_like(s1_ref)
        q1_ref[...] = jnp.zeros_like(q1_ref)
        s2_ref[...] = jnp.zeros_like(s2_ref)
        q2_ref[...] = jnp.zeros_like(q2_ref)
        p_ref[...] = jnp.zeros_like(p_ref)

    z1 = f1_ref[...]
    z2 = f2_ref[...]
    lg = lg_ref[...]
    yb = y_ref[...]

    bB = z1.shape[0]
    ks = jax.lax.broadcasted_iota(jnp.int32, (bB, KPAD), 1)
    oh = (yb == ks).astype(jnp.float32)

    dn = (((0,), (0,)), ((), ()))
    cr_ref[...] += jnp.sum(oh, axis=0, keepdims=True)
    cc_ref[...] += jax.lax.dot_general(
        oh, jnp.ones((bB, 1), jnp.float32), dn, preferred_element_type=jnp.float32)
    s1_ref[...] += jax.lax.dot_general(oh, z1, dn, preferred_element_type=jnp.float32)
    q1_ref[...] += jax.lax.dot_general(oh, z1 * z1, dn, preferred_element_type=jnp.float32)
    s2_ref[...] += jax.lax.dot_general(oh, z2, dn, preferred_element_type=jnp.float32)
    q2_ref[...] += jax.lax.dot_general(oh, z2 * z2, dn, preferred_element_type=jnp.float32)

    m = jnp.max(lg, axis=1, keepdims=True)
    e = jnp.exp(lg - m)
    p = e / jnp.sum(e, axis=1, keepdims=True)
    p_ref[...] += jax.lax.dot_general(oh, p, dn, preferred_element_type=jnp.float32)

    @pl.when(step == nsteps - 1)
    def _finale():
        K = p_ref.shape[1]
        c_row = cr_ref[...]
        c_col = cc_ref[...]
        rinv = 1.0 / jnp.maximum(c_col, 1.0)

        valid_row = (c_row >= 2.0)
        valid_col = (c_col >= 2.0)
        num_valid = jnp.sum(valid_row.astype(jnp.float32))
        kept = jnp.sum(jnp.where(valid_row, c_row, 0.0))
        layer_valid = jnp.logical_and(num_valid >= 2.0, kept >= 4.0)

        mu1 = s1_ref[...] * rinv
        var1 = jnp.maximum(q1_ref[...] * rinv - mu1 * mu1, EPS)
        mu2 = s2_ref[...] * rinv
        var2 = jnp.maximum(q2_ref[...] * rinv - mu2 * mu2, EPS)

        mean_p = p_ref[...] * rinv
        ei = jax.lax.broadcasted_iota(jnp.int32, (K, KPAD), 0)
        ej = jax.lax.broadcasted_iota(jnp.int32, (K, KPAD), 1)
        pad_eye = (ei == ej).astype(jnp.float32)
        mp = jax.lax.dot_general(mean_p, pad_eye, (((1,), (0,)), ((), ())),
                                 preferred_element_type=jnp.float32)
        alpha = 0.5 * (mp + mp.T)

        ri = jax.lax.broadcasted_iota(jnp.int32, (KPAD, KPAD), 0)
        cj = jax.lax.broadcasted_iota(jnp.int32, (KPAD, KPAD), 1)
        keep = jnp.logical_and(cj > ri, jnp.logical_and(valid_col, valid_row))
        amat = jnp.where(keep, alpha, 0.0)
        fidx = ri * KPAD + cj
        fidxT = cj * KPAD + ri

        for p_ in range(TOP_M):
            a = jnp.max(amat, axis=(0, 1), keepdims=True)
            idx = jnp.min(jnp.where(amat == a, fidx, jnp.int32(2 ** 30)),
                          axis=(0, 1), keepdims=True)
            mask = fidx == idx
            jind = jnp.sum(mask.astype(jnp.float32), axis=0, keepdims=True)
            iind = jnp.sum((fidxT == idx).astype(jnp.float32), axis=0,
                           keepdims=True)
            seli_ref[pl.ds(p_, 1), :] = iind
            selj_ref[pl.ds(p_, 1), :] = jind
            avec_ref[pl.ds(p_, 1), :] = a
            amat = jnp.where(mask, -1.0, amat)

        sel_i = seli_ref[...]
        sel_j = selj_ref[...]
        sel_d = sel_i - sel_j
        avec = avec_ref[...]
        dnm = (((1,), (0,)), ((), ()))

        def layer(mu, var):
            d = jax.lax.dot_general(sel_d, mu, dnm, preferred_element_type=jnp.float32)
            vi = jax.lax.dot_general(sel_i, var, dnm, preferred_element_type=jnp.float32)
            vj = jax.lax.dot_general(sel_j, var, dnm, preferred_element_type=jnp.float32)
            va = 0.5 * (vi + vj) + EPS
            t1 = 0.125 * jnp.sum(d * d / va, axis=1, keepdims=True)
            t2 = 0.25 * jnp.sum(
                jnp.log(va * va / ((vi + EPS) * (vj + EPS))), axis=1, keepdims=True)
            dm = jnp.maximum(t1 + t2, 0.0)
            rho = jnp.exp(-dm)
            return jnp.sum(avec * rho)

        num1 = layer(mu1, var1)
        num2 = layer(mu2, var2)
        den = jnp.maximum(jnp.sum(avec), EPS)
        total = (num1 + num2) / den
        out_ref[...] = jnp.full((1, 1), jnp.where(layer_valid, total * 0.5, 0.0),
                                jnp.float32)


def kernel(feat_layer1, feat_layer2, logits, y):
    B, D = feat_layer1.shape
    K = logits.shape[1]
    bB = 2048
    grid = B // bB

    y2 = y.astype(jnp.int32).reshape(B, 1)

    out = pl.pallas_call(
        _fused,
        grid=(grid,),
        in_specs=[
            pl.BlockSpec((bB, D), lambda i: (i, 0)),
            pl.BlockSpec((bB, D), lambda i: (i, 0)),
            pl.BlockSpec((bB, K), lambda i: (i, 0)),
            pl.BlockSpec((bB, 1), lambda i: (i, 0)),
        ],
        out_specs=pl.BlockSpec((1, 1), lambda i: (0, 0)),
        out_shape=jax.ShapeDtypeStruct((1, 1), jnp.float32),
        scratch_shapes=[
            pltpu.VMEM((1, KPAD), jnp.float32),
            pltpu.VMEM((KPAD, 1), jnp.float32),
            pltpu.VMEM((KPAD, D), jnp.float32),
            pltpu.VMEM((KPAD, D), jnp.float32),
            pltpu.VMEM((KPAD, D), jnp.float32),
            pltpu.VMEM((KPAD, D), jnp.float32),
            pltpu.VMEM((KPAD, K), jnp.float32),
            pltpu.VMEM((TOP_M, KPAD), jnp.float32),
            pltpu.VMEM((TOP_M, KPAD), jnp.float32),
            pltpu.VMEM((TOP_M, 1), jnp.float32),
        ],
    )(feat_layer1.astype(jnp.float32), feat_layer2.astype(jnp.float32),
      logits.astype(jnp.float32), y2)
    return out.reshape(())
